# baseline (speedup 1.0000x reference)
import jax
import jax.numpy as jnp
from jax import lax
from jax.experimental import pallas as pl
from jax.experimental.pallas import tpu as pltpu
from jax.experimental.pallas import tpu_sc as plsc

_B = 512
_BS = 19
_P = _BS * _BS
_MAXM = _P
_EMPTY = -1.0
_LANES = 16
_NCH = _B // _LANES
_RTW = 40
_RTOFF_MAX = 328


_MB = 19
_NMB = (_MAXM + _MB - 1) // _MB


def _tc_hist_body(mv_ref, board_ref, hist_ref, out_ref):
    m0 = pl.program_id(0) * _MB
    m_ids = m0 + lax.broadcasted_iota(jnp.int32, (_MB, 1, 1), 0)
    mv = mv_ref[...].reshape(1, 1, _B)
    sel = m_ids == mv
    board = board_ref[...].reshape(1, _P, _B)
    out_ref[...] = jnp.where(sel, board, hist_ref[...])


_tc_hist = pl.pallas_call(
    _tc_hist_body,
    grid=(_NMB,),
    in_specs=[
        pl.BlockSpec((1, _B), lambda i: (0, 0)),
        pl.BlockSpec((_P, _B), lambda i: (0, 0)),
        pl.BlockSpec((_MB, _P, _B), lambda i: (i, 0, 0)),
    ],
    out_specs=pl.BlockSpec((_MB, _P, _B), lambda i: (i, 0, 0)),
    out_shape=jax.ShapeDtypeStruct((_MAXM, _P, _B), jnp.float32),
    compiler_params=pltpu.CompilerParams(
        dimension_semantics=("arbitrary",),
    ),
    name="go_hist_tc",
)


def _sc_board_body(board_h, roots_h, pos_h, ply_h, cg_h,
                   board_out,
                   b_v, rt_v, cg_v, g_v, r_v, c_v, ply_v):
    wid = lax.axis_index("s") * 2 + lax.axis_index("c")

    @pl.when(wid < _BS)
    def _():
        w = wid
        pltpu.sync_copy(pos_h.at[0], r_v)
        pltpu.sync_copy(pos_h.at[1], c_v)
        pltpu.sync_copy(ply_h, ply_v)
        pltpu.sync_copy(board_h.at[w], b_v)
        off = jnp.minimum((w * _BS) // 8 * 8, _RTOFF_MAX)
        off = pl.multiple_of(off, 8)
        local_r = w * _BS - off
        pltpu.sync_copy(roots_h.at[pl.ds(off, _RTW)], rt_v)

        iota = lax.iota(jnp.int32, _LANES)

        def rr_body(rr, carry):
            pltpu.sync_copy(cg_h.at[rr], cg_v)

            def ch_body(ch, c2):
                l0 = ch * _LANES
                lanes = l0 + iota
                rv = r_v[pl.ds(l0, _LANES)]
                cv = c_v[pl.ds(l0, _LANES)]
                rc = jnp.clip(rv, 0, _BS - 1)
                cc = jnp.clip(cv, 0, _BS - 1)
                hit = rc == rr
                for k in range(4):
                    kf = jnp.full((_LANES,), k, jnp.int32)
                    val = plsc.load_gather(cg_v, [cc, kf, lanes])
                    cur = g_v[k, pl.ds(l0, _LANES)]
                    g_v[k, pl.ds(l0, _LANES)] = jnp.where(hit, val, cur)
                return c2

            return lax.fori_loop(0, _NCH, ch_body, carry)

        lax.fori_loop(0, _BS, rr_body, jnp.int32(0))

        def ch2_body(ch, carry):
            l0 = ch * _LANES
            rv = r_v[pl.ds(l0, _LANES)]
            cv = c_v[pl.ds(l0, _LANES)]
            rc = jnp.clip(rv, 0, _BS - 1)
            cc = jnp.clip(cv, 0, _BS - 1)
            play = (rv >= 0) & (cv >= 0)
            ply = ply_v[pl.ds(l0, _LANES)].astype(jnp.float32)
            g0 = g_v[0, pl.ds(l0, _LANES)]
            g1 = g_v[1, pl.ds(l0, _LANES)]
            g2 = g_v[2, pl.ds(l0, _LANES)]
            g3 = g_v[3, pl.ds(l0, _LANES)]
            place_row = play & (rc == w)

            def c_body(c, c2):
                bvals = b_v[c, pl.ds(l0, _LANES)]
                rtv = rt_v[local_r + c, pl.ds(l0, _LANES)]
                v = jnp.where(place_row & (cc == c), ply, bvals)
                cap = (((rtv == g0) & (g0 >= 0)) | ((rtv == g1) & (g1 >= 0))
                       | ((rtv == g2) & (g2 >= 0)) | ((rtv == g3) & (g3 >= 0)))
                v = jnp.where(play & cap, jnp.float32(_EMPTY), v)
                b_v[c, pl.ds(l0, _LANES)] = v
                return c2

            return lax.fori_loop(0, _BS, c_body, carry)

        lax.fori_loop(0, _NCH, ch2_body, jnp.int32(0))

        pltpu.sync_copy(b_v, board_out.at[w])


_mesh = plsc.VectorSubcoreMesh(core_axis_name="c", subcore_axis_name="s")

_sc_board = pl.kernel(
    _sc_board_body,
    out_type=jax.ShapeDtypeStruct((_BS, _BS, _B), jnp.float32),
    mesh=_mesh,
    scratch_types=[
        pltpu.VMEM((_BS, _B), jnp.float32),
        pltpu.VMEM((_RTW, _B), jnp.int32),
        pltpu.VMEM((_BS, 4, _B), jnp.int32),
        pltpu.VMEM((4, _B), jnp.int32),
        pltpu.VMEM((_B,), jnp.int32),
        pltpu.VMEM((_B,), jnp.int32),
        pltpu.VMEM((_B,), jnp.int32),
    ],
    compiler_params=pltpu.CompilerParams(needs_layout_passes=False),
    name="go_board_sc",
)


def kernel(board, board_history, positions, current_player, pass_count,
           move_count, roots, capture_groups):
    del pass_count
    hist_t = board_history.transpose(1, 2, 0)
    board_t = board.transpose(1, 2, 0)
    board_r = board_t.reshape(_P, _B)
    roots_t = roots.transpose(1, 0)
    pos_t = positions.transpose(1, 0)
    cg_t = capture_groups.transpose(1, 2, 3, 0)

    hist_out_t = _tc_hist(move_count.reshape(1, _B), board_r, hist_t)
    board_out_t = _sc_board(board_t, roots_t, pos_t, current_player, cg_t)
    return board_out_t.transpose(2, 0, 1), hist_out_t.transpose(2, 0, 1)
